# baseline (speedup 1.0000x reference)
import functools

import jax
import jax.numpy as jnp
from jax import lax
from jax.experimental import pallas as pl
from jax.experimental.pallas import tpu as pltpu
from jax.experimental.pallas import tpu_sc as plsc

D = 128
DW = D // 2
E = 320000
NC, NS, L = 2, 16, 16
NW = NC * NS
PER_W = E // NW
C = 80
NCHUNK = PER_W // C
NPAIR = (NCHUNK - 1) // 2


def _sc_kernel(x1_hbm, x2_hbm, i1_hbm, i2_hbm, out_hbm,
               idx1_v, idx2_v, rA1, rA2, rB1, rB2, psum_v, outw_v,
               si1, si2, sA1, sA2, sB1, sB2):
  wid = lax.axis_index("s") * NC + lax.axis_index("c")
  wbase = wid * PER_W
  lane16 = lax.iota(jnp.int32, L) * L

  cpi1 = pltpu.async_copy(i1_hbm.at[pl.ds(wbase, PER_W)], idx1_v, si1)
  cpi2 = pltpu.async_copy(i2_hbm.at[pl.ds(wbase, PER_W)], idx2_v, si2)
  cpi1.wait()
  cpi2.wait()

  def start(c, r1, r2, s1, s2):
    pltpu.async_copy(x1_hbm.at[idx1_v.at[pl.ds(c * C, C)]], r1, s1)
    pltpu.async_copy(x2_hbm.at[idx2_v.at[pl.ds(c * C, C)]], r2, s2)

  def wait(c, r1, r2, s1, s2):
    pltpu.make_async_copy(x1_hbm.at[idx1_v.at[pl.ds(c * C, C)]], r1, s1).wait()
    pltpu.make_async_copy(x2_hbm.at[idx2_v.at[pl.ds(c * C, C)]], r2, s2).wait()

  def compute(c, r1, r2):
    def group_body(g, carry):
      for j in range(L):
        acc = jnp.zeros((L,), jnp.float32)
        for k in range(DW // L):
          ab1 = plsc.bitcast(r1[g * L + j, pl.ds(k * L, L)], jnp.bfloat16)
          ab2 = plsc.bitcast(r2[g * L + j, pl.ds(k * L, L)], jnp.bfloat16)
          a0, a1 = plsc.unpack(ab1, format=plsc.PackFormat.INTERLEAVED)
          b0, b1 = plsc.unpack(ab2, format=plsc.PackFormat.INTERLEAVED)
          acc = acc + a0 * b0
          acc = acc + a1 * b1
        psum_v[pl.ds(j * L, L)] = acc
      tot = plsc.load_gather(psum_v, [lane16])
      for l in range(1, L):
        tot = tot + plsc.load_gather(psum_v, [lane16 + l])
      outw_v[pl.ds(c * C + g * L, L)] = tot
      return carry
    lax.fori_loop(0, C // L, group_body, 0)

  start(0, rA1, rA2, sA1, sA2)

  def pair_body(i, carry):
    c0 = 2 * i
    start(c0 + 1, rB1, rB2, sB1, sB2)
    wait(c0, rA1, rA2, sA1, sA2)
    compute(c0, rA1, rA2)
    start(c0 + 2, rA1, rA2, sA1, sA2)
    wait(c0 + 1, rB1, rB2, sB1, sB2)
    compute(c0 + 1, rB1, rB2)
    return carry

  lax.fori_loop(0, NPAIR, pair_body, 0)
  wait(NCHUNK - 1, rA1, rA2, sA1, sA2)
  compute(NCHUNK - 1, rA1, rA2)

  pltpu.sync_copy(outw_v, out_hbm.at[pl.ds(wbase, PER_W)])


@functools.partial(
    pl.kernel,
    mesh=plsc.VectorSubcoreMesh(core_axis_name="c", subcore_axis_name="s"),
    out_type=jax.ShapeDtypeStruct((E,), jnp.float32),
    compiler_params=pltpu.CompilerParams(needs_layout_passes=False,
                                         use_tc_tiling_on_sc=False),
    scratch_types=[
        pltpu.VMEM((PER_W,), jnp.int32),
        pltpu.VMEM((PER_W,), jnp.int32),
        pltpu.VMEM((C, DW), jnp.int32),
        pltpu.VMEM((C, DW), jnp.int32),
        pltpu.VMEM((C, DW), jnp.int32),
        pltpu.VMEM((C, DW), jnp.int32),
        pltpu.VMEM((L * L,), jnp.float32),
        pltpu.VMEM((PER_W,), jnp.float32),
        pltpu.SemaphoreType.DMA,
        pltpu.SemaphoreType.DMA,
        pltpu.SemaphoreType.DMA,
        pltpu.SemaphoreType.DMA,
        pltpu.SemaphoreType.DMA,
        pltpu.SemaphoreType.DMA,
    ],
)
def _edge_dot(x1, x2, i1, i2, out, *scratch):
  _sc_kernel(x1, x2, i1, i2, out, *scratch)


def _pack_table(x):
  xb = x.astype(jnp.bfloat16).reshape(x.shape[0], x.shape[1] // 2, 2)
  return lax.bitcast_convert_type(xb, jnp.int32)


def kernel(x_nt1, x_nt2, edge_label_index):
  i1 = edge_label_index[0].astype(jnp.int32)
  i2 = edge_label_index[1].astype(jnp.int32)
  return _edge_dot(_pack_table(x_nt1), _pack_table(x_nt2), i1, i2)


# speedup vs baseline: 1.2706x; 1.2706x over previous
import functools

import jax
import jax.numpy as jnp
from jax import lax
from jax.experimental import pallas as pl
from jax.experimental.pallas import tpu as pltpu
from jax.experimental.pallas import tpu_sc as plsc

D = 128
DW = D // 2
E = 320000
NC, NS, L = 2, 16, 16
NW = NC * NS
PER_W = E // NW
C = 80
NCHUNK = PER_W // C
NPAIR = (NCHUNK - 1) // 2


def _sc_kernel(x1_hbm, x2_hbm, i_hbm, out_hbm,
               idx1_v, idx2_v, rA1, rA2, rB1, rB2, psum_v, outw_v,
               si1, si2, sA1, sA2, sB1, sB2):
  wid = lax.axis_index("s") * NC + lax.axis_index("c")
  wbase = wid * PER_W
  lane16 = lax.iota(jnp.int32, L) * L

  cpi1 = pltpu.async_copy(i_hbm.at[0, pl.ds(wbase, PER_W)], idx1_v, si1)
  cpi2 = pltpu.async_copy(i_hbm.at[1, pl.ds(wbase, PER_W)], idx2_v, si2)
  cpi1.wait()
  cpi2.wait()

  def start(c, r1, r2, s1, s2):
    pltpu.async_copy(x1_hbm.at[idx1_v.at[pl.ds(c * C, C)]], r1, s1)
    pltpu.async_copy(x2_hbm.at[idx2_v.at[pl.ds(c * C, C)]], r2, s2)

  def wait(c, r1, r2, s1, s2):
    pltpu.make_async_copy(x1_hbm.at[idx1_v.at[pl.ds(c * C, C)]], r1, s1).wait()
    pltpu.make_async_copy(x2_hbm.at[idx2_v.at[pl.ds(c * C, C)]], r2, s2).wait()

  def compute(c, r1, r2):
    def group_body(g, carry):
      for j in range(L):
        acc = jnp.zeros((L,), jnp.float32)
        for k in range(DW // L):
          ab1 = plsc.bitcast(r1[g * L + j, pl.ds(k * L, L)], jnp.bfloat16)
          ab2 = plsc.bitcast(r2[g * L + j, pl.ds(k * L, L)], jnp.bfloat16)
          a0, a1 = plsc.unpack(ab1, format=plsc.PackFormat.INTERLEAVED)
          b0, b1 = plsc.unpack(ab2, format=plsc.PackFormat.INTERLEAVED)
          acc = acc + a0 * b0
          acc = acc + a1 * b1
        psum_v[pl.ds(j * L, L)] = acc
      tot = plsc.load_gather(psum_v, [lane16])
      for l in range(1, L):
        tot = tot + plsc.load_gather(psum_v, [lane16 + l])
      outw_v[pl.ds(c * C + g * L, L)] = tot
      return carry
    lax.fori_loop(0, C // L, group_body, 0)

  start(0, rA1, rA2, sA1, sA2)

  def pair_body(i, carry):
    c0 = 2 * i
    start(c0 + 1, rB1, rB2, sB1, sB2)
    wait(c0, rA1, rA2, sA1, sA2)
    compute(c0, rA1, rA2)
    start(c0 + 2, rA1, rA2, sA1, sA2)
    wait(c0 + 1, rB1, rB2, sB1, sB2)
    compute(c0 + 1, rB1, rB2)
    return carry

  lax.fori_loop(0, NPAIR, pair_body, 0)
  wait(NCHUNK - 1, rA1, rA2, sA1, sA2)
  compute(NCHUNK - 1, rA1, rA2)

  pltpu.sync_copy(outw_v, out_hbm.at[pl.ds(wbase, PER_W)])


@functools.partial(
    pl.kernel,
    mesh=plsc.VectorSubcoreMesh(core_axis_name="c", subcore_axis_name="s"),
    out_type=jax.ShapeDtypeStruct((E,), jnp.float32),
    compiler_params=pltpu.CompilerParams(needs_layout_passes=False,
                                         use_tc_tiling_on_sc=False),
    scratch_types=[
        pltpu.VMEM((PER_W,), jnp.int32),
        pltpu.VMEM((PER_W,), jnp.int32),
        pltpu.VMEM((C, DW), jnp.uint32),
        pltpu.VMEM((C, DW), jnp.uint32),
        pltpu.VMEM((C, DW), jnp.uint32),
        pltpu.VMEM((C, DW), jnp.uint32),
        pltpu.VMEM((L * L,), jnp.float32),
        pltpu.VMEM((PER_W,), jnp.float32),
        pltpu.SemaphoreType.DMA,
        pltpu.SemaphoreType.DMA,
        pltpu.SemaphoreType.DMA,
        pltpu.SemaphoreType.DMA,
        pltpu.SemaphoreType.DMA,
        pltpu.SemaphoreType.DMA,
    ],
)
def _edge_dot(x1, x2, ei, out, *scratch):
  _sc_kernel(x1, x2, ei, out, *scratch)


def _pack_body(x1_ref, x2_ref, o1_ref, o2_ref):
  for x_ref, o_ref in ((x1_ref, o1_ref), (x2_ref, o2_ref)):
    x = x_ref[...]
    lo = lax.bitcast_convert_type(
        x[:, :DW].astype(jnp.bfloat16), jnp.uint16).astype(jnp.uint32)
    hi = lax.bitcast_convert_type(
        x[:, DW:].astype(jnp.bfloat16), jnp.uint16).astype(jnp.uint32)
    o_ref[...] = lo | (hi << 16)


def _pack_tables(x1, x2):
  n = x1.shape[0]
  return pl.pallas_call(
      _pack_body,
      out_shape=(jax.ShapeDtypeStruct((n, DW), jnp.uint32),
                 jax.ShapeDtypeStruct((n, DW), jnp.uint32)),
  )(x1, x2)


def kernel(x_nt1, x_nt2, edge_label_index):
  x1p, x2p = _pack_tables(x_nt1, x_nt2)
  return _edge_dot(x1p, x2p, edge_label_index.astype(jnp.int32))
